# Initial kernel scaffold; baseline (speedup 1.0000x reference)
import jax
import jax.numpy as jnp
from jax.experimental import pallas as pl


def kernel(keys, values, importance, query, mem):
    raise NotImplementedError("write your pallas kernel here")


# speedup vs baseline: 1.1964x; 1.1964x over previous
import functools
import math

import jax
import jax.numpy as jnp
from jax.experimental import pallas as pl
from jax.experimental.pallas import tpu as pltpu

_MOMENTUM = 0.99
_NEW_RATE = 0.01
_NEG_INF = float("-inf")


def _stats_kernel(keys_ref, mem_ref, m_ref, s_ref, m_sc, s_sc):
    i = pl.program_id(0)

    @pl.when(i == 0)
    def _init():
        m_sc[...] = jnp.full_like(m_sc, _NEG_INF)
        s_sc[...] = jnp.zeros_like(s_sc)

    sc = jnp.dot(keys_ref[...], mem_ref[...].T, preferred_element_type=jnp.float32)
    bm = jnp.max(sc, axis=1, keepdims=True)
    m_old = m_sc[...]
    m_new = jnp.maximum(m_old, bm)
    s_sc[...] = s_sc[...] * jnp.exp(m_old - m_new) + jnp.sum(
        jnp.exp(sc - m_new), axis=1, keepdims=True
    )
    m_sc[...] = m_new

    @pl.when(i == pl.num_programs(0) - 1)
    def _fin():
        m_ref[...] = m_sc[...]
        s_ref[...] = s_sc[...]


def _priority_kernel(b, k, keys_ref, mem_ref, m_ref, s_ref, imp_ref, out_ref):
    sc = jnp.dot(keys_ref[...], mem_ref[...].T, preferred_element_type=jnp.float32)
    e = jnp.exp(sc - m_ref[...]) / s_ref[...]
    cb = sc.shape[1]
    usage = e.reshape(b, k, cb).sum(axis=1)
    combined = jnp.mean(imp_ref[...], axis=1)
    out_ref[...] = combined + 0.1 * usage


def _topk_kernel(b, k, nchunk, p_ref, idx_ref, ps, cms):
    ps[...] = p_ref[...]
    cms[...] = jnp.max(ps[...], axis=2)
    iota_c = jax.lax.broadcasted_iota(jnp.int32, (1, nchunk), 1)
    iota_r = jax.lax.broadcasted_iota(jnp.int32, (1, 128), 1)

    def step(r, carry):
        for bb in range(b):
            cm = cms[pl.ds(bb, 1), :]
            c = jnp.argmax(cm)
            row = ps[bb, pl.ds(c, 1), :]
            pos = jnp.argmax(row)
            idxv = (c * 128 + pos).astype(jnp.int32)
            old = idx_ref[pl.ds(bb, 1), :]
            idx_ref[pl.ds(bb, 1), :] = jnp.where(iota_r == r, idxv, old)
            nrow = jnp.where(iota_r == pos, _NEG_INF, row)
            ps[bb, pl.ds(c, 1), :] = nrow
            cms[pl.ds(bb, 1), :] = jnp.where(iota_c == c, jnp.max(nrow), cm)
        return carry

    jax.lax.fori_loop(0, k, step, 0)


def _scatter_kernel(b, k, scale, idx_ref, mem_ref, vals_ref, out_ref):
    out_ref[...] = mem_ref[...] * _MOMENTUM

    for bb in range(b):
        def body(i, carry, bb=bb):
            j = idx_ref[bb, i]
            out_ref[pl.ds(j, 1), :] = (
                out_ref[pl.ds(j, 1), :] + vals_ref[bb, pl.ds(i, 1), :] * scale
            )
            return carry

        jax.lax.fori_loop(0, k, body, 0)


def _read_stats_kernel(inv_sqrt_d, q_ref, kv_ref, m_ref, s_ref, m_sc, s_sc):
    i = pl.program_id(0)

    @pl.when(i == 0)
    def _init():
        m_sc[...] = jnp.full_like(m_sc, _NEG_INF)
        s_sc[...] = jnp.zeros_like(s_sc)

    l = jnp.dot(q_ref[...], kv_ref[...].T, preferred_element_type=jnp.float32) * inv_sqrt_d
    bm = jnp.max(l, axis=1, keepdims=True)
    m_old = m_sc[...]
    m_new = jnp.maximum(m_old, bm)
    s_sc[...] = s_sc[...] * jnp.exp(m_old - m_new) + jnp.sum(
        jnp.exp(l - m_new), axis=1, keepdims=True
    )
    m_sc[...] = m_new

    @pl.when(i == pl.num_programs(0) - 1)
    def _fin():
        m_ref[...] = m_sc[...]
        s_ref[...] = s_sc[...]


def _read_out_kernel(inv_sqrt_d, q_ref, kv_ref, m_ref, s_ref, o_ref, acc):
    i = pl.program_id(0)

    @pl.when(i == 0)
    def _init():
        acc[...] = jnp.zeros_like(acc)

    l = jnp.dot(q_ref[...], kv_ref[...].T, preferred_element_type=jnp.float32) * inv_sqrt_d
    w = jnp.exp(l - m_ref[...]) / s_ref[...]
    acc[...] += jnp.dot(w, kv_ref[...], preferred_element_type=jnp.float32)

    @pl.when(i == pl.num_programs(0) - 1)
    def _fin():
        o_ref[...] = acc[...]


def kernel(keys, values, importance, query, mem):
    b, k_orig, d = keys.shape
    cap = mem.shape[1]
    q = query.shape[1]
    k = min(k_orig, cap)
    bk = b * k_orig
    bq = b * q

    mem2 = mem.reshape(cap, d)
    keys2 = keys.reshape(bk, d)
    imp2 = importance.reshape(b, -1, cap)
    nplane = imp2.shape[1]
    query2 = query.reshape(bq, d)

    cb = 2048
    grid = cap // cb

    m_rows, s_rows = pl.pallas_call(
        _stats_kernel,
        grid=(grid,),
        in_specs=[
            pl.BlockSpec((bk, d), lambda i: (0, 0)),
            pl.BlockSpec((cb, d), lambda i: (i, 0)),
        ],
        out_specs=[
            pl.BlockSpec((bk, 1), lambda i: (0, 0)),
            pl.BlockSpec((bk, 1), lambda i: (0, 0)),
        ],
        out_shape=[
            jax.ShapeDtypeStruct((bk, 1), jnp.float32),
            jax.ShapeDtypeStruct((bk, 1), jnp.float32),
        ],
        scratch_shapes=[
            pltpu.VMEM((bk, 1), jnp.float32),
            pltpu.VMEM((bk, 1), jnp.float32),
        ],
    )(keys2, mem2)

    priority = pl.pallas_call(
        functools.partial(_priority_kernel, b, k_orig),
        grid=(grid,),
        in_specs=[
            pl.BlockSpec((bk, d), lambda i: (0, 0)),
            pl.BlockSpec((cb, d), lambda i: (i, 0)),
            pl.BlockSpec((bk, 1), lambda i: (0, 0)),
            pl.BlockSpec((bk, 1), lambda i: (0, 0)),
            pl.BlockSpec((b, nplane, cb), lambda i: (0, 0, i)),
        ],
        out_specs=pl.BlockSpec((b, cb), lambda i: (0, i)),
        out_shape=jax.ShapeDtypeStruct((b, cap), jnp.float32),
    )(keys2, mem2, m_rows, s_rows, imp2)

    nchunk = cap // 128
    p3 = priority.reshape(b, nchunk, 128)
    indices = pl.pallas_call(
        functools.partial(_topk_kernel, b, k, nchunk),
        in_specs=[pl.BlockSpec((b, nchunk, 128), lambda: (0, 0, 0))],
        out_specs=pl.BlockSpec((b, 128), lambda: (0, 0)),
        out_shape=jax.ShapeDtypeStruct((b, 128), jnp.int32),
        scratch_shapes=[
            pltpu.VMEM((b, nchunk, 128), jnp.float32),
            pltpu.VMEM((b, nchunk), jnp.float32),
        ],
    )(p3)

    scale = _NEW_RATE / b
    new_mem = pl.pallas_call(
        functools.partial(_scatter_kernel, b, k, scale),
        in_specs=[
            pl.BlockSpec(memory_space=pltpu.SMEM),
            pl.BlockSpec((cap, d), lambda: (0, 0)),
            pl.BlockSpec((b, k, d), lambda: (0, 0, 0)),
        ],
        out_specs=pl.BlockSpec((cap, d), lambda: (0, 0)),
        out_shape=jax.ShapeDtypeStruct((cap, d), jnp.float32),
    )(indices, mem2, values[:, :k])

    fb = 1024
    rgrid = cap // fb
    isd = 1.0 / math.sqrt(d)
    m_q, s_q = pl.pallas_call(
        functools.partial(_read_stats_kernel, isd),
        grid=(rgrid,),
        in_specs=[
            pl.BlockSpec((bq, d), lambda i: (0, 0)),
            pl.BlockSpec((fb, d), lambda i: (i, 0)),
        ],
        out_specs=[
            pl.BlockSpec((bq, 1), lambda i: (0, 0)),
            pl.BlockSpec((bq, 1), lambda i: (0, 0)),
        ],
        out_shape=[
            jax.ShapeDtypeStruct((bq, 1), jnp.float32),
            jax.ShapeDtypeStruct((bq, 1), jnp.float32),
        ],
        scratch_shapes=[
            pltpu.VMEM((bq, 1), jnp.float32),
            pltpu.VMEM((bq, 1), jnp.float32),
        ],
    )(query2, new_mem)

    out = pl.pallas_call(
        functools.partial(_read_out_kernel, isd),
        grid=(rgrid,),
        in_specs=[
            pl.BlockSpec((bq, d), lambda i: (0, 0)),
            pl.BlockSpec((fb, d), lambda i: (i, 0)),
            pl.BlockSpec((bq, 1), lambda i: (0, 0)),
            pl.BlockSpec((bq, 1), lambda i: (0, 0)),
        ],
        out_specs=pl.BlockSpec((bq, d), lambda i: (0, 0)),
        out_shape=jax.ShapeDtypeStruct((bq, d), jnp.float32),
        scratch_shapes=[pltpu.VMEM((bq, d), jnp.float32)],
    )(query2, new_mem, m_q, s_q)

    return out.reshape(b, q, d)
